# baseline (speedup 1.0000x reference)
import functools

import numpy as np
import jax
import jax.numpy as jnp
from jax import lax
from jax.experimental import pallas as pl
from jax.experimental.pallas import tpu as pltpu
from jax.experimental.pallas import tpu_sc as plsc

NV = 6890
NPAD = 6912
NEDGE = 6890 * 8
NB = 2
EPS = 1e-5
CNT = 8.0 * NV

_F32 = jnp.float32

_GS_NP = {}
for _c in (16, 32, 64, 128):
    _g = np.zeros((_c // 8, _c), np.float32)
    for _i in range(_c // 8):
        _g[_i, 8 * _i:8 * _i + 8] = 1.0
    _GS_NP[_c] = _g

_MASK_NP = np.zeros((1, NPAD), np.float32)
_MASK_NP[0, :NV] = 1.0


def _mm(a, b):
    return lax.dot_general(a, b, (((1,), (0,)), ((), ())),
                           preferred_element_type=_F32,
                           precision=lax.Precision.HIGHEST)


def _gn_relu(x, gvec, bvec, mask, gnm):
    c = x.shape[0]
    gs = gnm['gs%d' % c]
    et = gnm['et%d' % c]
    s1 = jnp.sum(x, axis=1, keepdims=True)
    m = _mm(gs, s1) / CNT
    d = (x - _mm(et, m)) * mask
    s2 = jnp.sum(d * d, axis=1, keepdims=True)
    v = _mm(gs, s2) / CNT
    inv_c = _mm(et, lax.rsqrt(v + EPS))
    return jnp.maximum(d * (gvec * inv_c) + bvec, 0.0) * mask


def _part_a(x, pa, mask, gnm):
    y = _gn_relu(x, pa['pre_g'], pa['pre_b'], mask, gnm)
    y = (_mm(pa['lin1_W'], y) + pa['lin1_b']) * mask
    y = _gn_relu(y, pa['n1_g'], pa['n1_b'], mask, gnm)
    return _mm(pa['conv_Wt'], y)


def _part_b(x, g, pb, mask, gnm):
    t = (g + pb['conv_b']) * mask
    z = _gn_relu(t, pb['n2_g'], pb['n2_b'], mask, gnm)
    y2 = (_mm(pb['lin2_W'], z) + pb['lin2_b']) * mask
    if 'skip_W' in pb:
        x = (_mm(pb['skip_W'], x) + pb['skip_b']) * mask
    return x + y2


def _run_tc(fn, inputs, out_shapes):
    flat, tdef = jax.tree_util.tree_flatten(inputs)
    n_in = len(flat)

    def body(*refs):
        ins = jax.tree_util.tree_unflatten(tdef, refs[:n_in])
        fn(ins, refs[n_in:])

    return pl.pallas_call(
        body,
        out_shape=[jax.ShapeDtypeStruct(s, _F32) for s in out_shapes],
    )(*flat)


def _k0_fn(ins, outs):
    mask = ins['mask'][...]
    gnm = {k: ins['gnm'][k][...] for k in ins['gnm']}
    refpart = _mm(ins['Wref'][...], ins['refp'][...])
    imgproj = lax.dot_general(ins['Wimg'][...], ins['img'][...],
                              (((1,), (1,)), ((), ())),
                              preferred_element_type=_F32,
                              precision=lax.Precision.HIGHEST)
    pa = {k: ins['pa'][k][...] for k in ins['pa']}
    for b in range(NB):
        x0 = (refpart + imgproj[:, b:b + 1] + ins['b0'][...]) * mask
        outs[0][b] = x0
        outs[1][b] = _part_a(x0, pa, mask, gnm)


def _kmid_fn(ins, outs):
    mask = ins['mask'][...]
    gnm = {k: ins['gnm'][k][...] for k in ins['gnm']}
    pb = {k: ins['pb'][k][...] for k in ins['pb']}
    pa = {k: ins['pa'][k][...] for k in ins['pa']}
    for b in range(NB):
        xk = _part_b(ins['x'][b], ins['g'][b], pb, mask, gnm)
        outs[0][b] = xk
        outs[1][b] = _part_a(xk, pa, mask, gnm)


def _kfin_fn(ins, outs):
    mask = ins['mask'][...]
    gnm = {k: ins['gnm'][k][...] for k in ins['gnm']}
    pb = {k: ins['pb'][k][...] for k in ins['pb']}
    for b in range(NB):
        xk = _part_b(ins['x'][b], ins['g'][b], pb, mask, gnm)
        z = _gn_relu(xk, ins['gn_g'][...], ins['gn_b'][...], mask, gnm)
        outs[0][b] = _mm(ins['Wout'][...], z) + ins['bout'][...]


_CHUNK = 11024
_NCHUNK = NEDGE // _CHUNK


@functools.cache
def _make_spmm(mid):
    cpt = mid // 16
    chunk = _CHUNK
    mesh = plsc.VectorSubcoreMesh(core_axis_name="c", subcore_axis_name="s",
                                  num_cores=2, num_subcores=16)

    @functools.partial(
        pl.kernel,
        out_type=jax.ShapeDtypeStruct((NB * mid * NPAD,), _F32),
        mesh=mesh,
        compiler_params=pltpu.CompilerParams(needs_layout_passes=False),
        scratch_types=[
            pltpu.VMEM((cpt * NPAD,), _F32),
            pltpu.VMEM((cpt * NPAD,), _F32),
            pltpu.VMEM((3 * chunk,), jnp.int32),
            pltpu.VMEM((3 * chunk,), jnp.int32),
            pltpu.SemaphoreType.DMA,
            pltpu.SemaphoreType.DMA,
            pltpu.SemaphoreType.DMA,
        ],
    )
    def spmm(s_hbm, pk_hbm, zero_hbm, out_hbm,
             sup_v, agg_v, eb0, eb1, sem0, sem1, semz):
        b = lax.axis_index("c")
        t = lax.axis_index("s")
        base = (b * mid + t * cpt) * NPAD
        ebufs = (eb0, eb1)
        sems = (sem0, sem1)
        cps = pltpu.async_copy(s_hbm.at[pl.ds(base, cpt * NPAD)], sup_v, semz)
        cpz = pltpu.async_copy(zero_hbm.at[pl.ds(0, cpt * NPAD)], agg_v, semz)
        cp = [None] * _NCHUNK
        cp[0] = pltpu.async_copy(pk_hbm.at[pl.ds(0, 3 * chunk)], eb0, sem0)
        cps.wait()
        cpz.wait()

        for c in range(_NCHUNK):
            if c + 1 < _NCHUNK:
                cp[c + 1] = pltpu.async_copy(
                    pk_hbm.at[pl.ds((c + 1) * 3 * chunk, 3 * chunk)],
                    ebufs[(c + 1) % 2], sems[(c + 1) % 2])
            cp[c].wait()
            eb = ebufs[c % 2]

            @plsc.parallel_loop(0, chunk // 16, 1, unroll=8)
            def ebody(i, eb=eb):
                sl = pl.ds(i * 16, 16)
                si = eb[sl]
                di = eb[pl.ds(chunk + i * 16, 16)]
                wv = plsc.bitcast(eb[pl.ds(2 * chunk + i * 16, 16)], _F32)
                for ch in range(cpt):
                    off = ch * NPAD
                    vals = plsc.load_gather(sup_v, [si + off]) * wv
                    plsc.addupdate_scatter(agg_v, [di + off], vals)

        pltpu.sync_copy(agg_v, out_hbm.at[pl.ds(base, cpt * NPAD)])

    return spmm


def _prep_a(p):
    return dict(pre_g=p['pre_g'][:, None], pre_b=p['pre_b'][:, None],
                lin1_W=p['lin1_W'], lin1_b=p['lin1_b'][:, None],
                n1_g=p['n1_g'][:, None], n1_b=p['n1_b'][:, None],
                conv_Wt=p['conv_W'].T)


def _prep_b(p):
    d = dict(conv_b=p['conv_b'][:, None],
             n2_g=p['n2_g'][:, None], n2_b=p['n2_b'][:, None],
             lin2_W=p['lin2_W'], lin2_b=p['lin2_b'][:, None])
    if 'skip_W' in p:
        d['skip_W'] = p['skip_W']
        d['skip_b'] = p['skip_b'][:, None]
    return d


def kernel(image_enc, ref_vertices, edge_w, params, edge_src, edge_dst):
    mask = jnp.asarray(_MASK_NP)
    gnm = {}
    for c in (16, 32, 64, 128):
        gnm['gs%d' % c] = jnp.asarray(_GS_NP[c])
        gnm['et%d' % c] = jnp.asarray(_GS_NP[c].T)
    blocks = list(params['blocks']) + list(params['shape_blocks'])

    refp = jnp.zeros((8, NPAD), _F32).at[:3, :NV].set(ref_vertices)
    w_ref = jnp.zeros((128, 8), _F32).at[:, :3].set(params['lin0_W'][:, :3])
    w_img = params['lin0_W'][:, 3:]

    x, s = _run_tc(
        _k0_fn,
        dict(img=image_enc, refp=refp, Wref=w_ref, Wimg=w_img,
             b0=params['lin0_b'][:, None], mask=mask, gnm=gnm,
             pa=_prep_a(blocks[0])),
        [(NB, 128, NPAD), (NB, 32, NPAD)],
    )

    wbits = lax.bitcast_convert_type(edge_w, jnp.int32)
    pk = jnp.stack([edge_src.reshape(_NCHUNK, _CHUNK),
                    edge_dst.reshape(_NCHUNK, _CHUNK),
                    wbits.reshape(_NCHUNK, _CHUNK)], axis=1).reshape(-1)
    zero_buf = jnp.zeros((2 * NPAD,), _F32)

    for k in range(8):
        mid = blocks[k]['conv_W'].shape[0]
        g = _make_spmm(mid)(s.reshape(-1), pk, zero_buf)
        g = g.reshape(NB, mid, NPAD)
        if k < 7:
            mid_next = blocks[k + 1]['conv_W'].shape[0]
            out_c = blocks[k]['lin2_W'].shape[0]
            x, s = _run_tc(
                _kmid_fn,
                dict(x=x, g=g, mask=mask, gnm=gnm,
                     pb=_prep_b(blocks[k]), pa=_prep_a(blocks[k + 1])),
                [(NB, out_c, NPAD), (NB, mid_next, NPAD)],
            )
        else:
            out, = _run_tc(
                _kfin_fn,
                dict(x=x, g=g, mask=mask, gnm=gnm, pb=_prep_b(blocks[k]),
                     gn_g=params['gn_g'][:, None], gn_b=params['gn_b'][:, None],
                     Wout=params['lin_out_W'], bout=params['lin_out_b'][:, None]),
                [(NB, 3, NPAD)],
            )

    return jnp.transpose(out, (0, 2, 1))[:, :NV, :]


# speedup vs baseline: 1.0033x; 1.0033x over previous
import functools

import numpy as np
import jax
import jax.numpy as jnp
from jax import lax
from jax.experimental import pallas as pl
from jax.experimental.pallas import tpu as pltpu
from jax.experimental.pallas import tpu_sc as plsc

NV = 6890
NPAD = 6912
NEDGE = 6890 * 8
NB = 2
EPS = 1e-5
CNT = 8.0 * NV

_F32 = jnp.float32

_GS_NP = {}
for _c in (16, 32, 64, 128):
    _g = np.zeros((_c // 8, _c), np.float32)
    for _i in range(_c // 8):
        _g[_i, 8 * _i:8 * _i + 8] = 1.0
    _GS_NP[_c] = _g

_MASK_NP = np.zeros((1, NPAD), np.float32)
_MASK_NP[0, :NV] = 1.0


def _mm(a, b):
    return lax.dot_general(a, b, (((1,), (0,)), ((), ())),
                           preferred_element_type=_F32,
                           precision=lax.Precision.HIGHEST)


def _gn_relu(x, gvec, bvec, mask, gnm):
    c = x.shape[0]
    gs = gnm['gs%d' % c]
    et = gnm['et%d' % c]
    s1 = jnp.sum(x, axis=1, keepdims=True)
    m = _mm(gs, s1) / CNT
    d = (x - _mm(et, m)) * mask
    s2 = jnp.sum(d * d, axis=1, keepdims=True)
    v = _mm(gs, s2) / CNT
    inv_c = _mm(et, 1.0 / jnp.sqrt(v + EPS))
    return jnp.maximum(d * (gvec * inv_c) + bvec, 0.0) * mask


def _part_a(x, pa, mask, gnm):
    y = _gn_relu(x, pa['pre_g'], pa['pre_b'], mask, gnm)
    y = (_mm(pa['lin1_W'], y) + pa['lin1_b']) * mask
    y = _gn_relu(y, pa['n1_g'], pa['n1_b'], mask, gnm)
    return _mm(pa['conv_Wt'], y)


def _part_b(x, g, pb, mask, gnm):
    t = (g + pb['conv_b']) * mask
    z = _gn_relu(t, pb['n2_g'], pb['n2_b'], mask, gnm)
    y2 = (_mm(pb['lin2_W'], z) + pb['lin2_b']) * mask
    if 'skip_W' in pb:
        x = (_mm(pb['skip_W'], x) + pb['skip_b']) * mask
    return x + y2


def _run_tc(fn, inputs, out_shapes):
    flat, tdef = jax.tree_util.tree_flatten(inputs)
    n_in = len(flat)

    def body(*refs):
        ins = jax.tree_util.tree_unflatten(tdef, refs[:n_in])
        fn(ins, refs[n_in:])

    return pl.pallas_call(
        body,
        out_shape=[jax.ShapeDtypeStruct(s, _F32) for s in out_shapes],
    )(*flat)


def _k0_fn(ins, outs):
    mask = ins['mask'][...]
    gnm = {k: ins['gnm'][k][...] for k in ins['gnm']}
    refpart = _mm(ins['Wref'][...], ins['refp'][...])
    imgproj = lax.dot_general(ins['Wimg'][...], ins['img'][...],
                              (((1,), (1,)), ((), ())),
                              preferred_element_type=_F32)
    pa = {k: ins['pa'][k][...] for k in ins['pa']}
    for b in range(NB):
        x0 = (refpart + imgproj[:, b:b + 1] + ins['b0'][...]) * mask
        outs[0][b] = x0
        outs[1][b] = _part_a(x0, pa, mask, gnm)


def _kmid_fn(ins, outs):
    mask = ins['mask'][...]
    gnm = {k: ins['gnm'][k][...] for k in ins['gnm']}
    pb = {k: ins['pb'][k][...] for k in ins['pb']}
    pa = {k: ins['pa'][k][...] for k in ins['pa']}
    for b in range(NB):
        xk = _part_b(ins['x'][b], ins['g'][b], pb, mask, gnm)
        outs[0][b] = xk
        outs[1][b] = _part_a(xk, pa, mask, gnm)


def _kfin_fn(ins, outs):
    mask = ins['mask'][...]
    gnm = {k: ins['gnm'][k][...] for k in ins['gnm']}
    pb = {k: ins['pb'][k][...] for k in ins['pb']}
    for b in range(NB):
        xk = _part_b(ins['x'][b], ins['g'][b], pb, mask, gnm)
        z = _gn_relu(xk, ins['gn_g'][...], ins['gn_b'][...], mask, gnm)
        outs[0][b] = _mm(ins['Wout'][...], z) + ins['bout'][...]


_CHUNK = 11024
_NCHUNK = NEDGE // _CHUNK


@functools.cache
def _make_spmm(mid):
    cpt = mid // 16
    chunk = _CHUNK
    mesh = plsc.VectorSubcoreMesh(core_axis_name="c", subcore_axis_name="s",
                                  num_cores=2, num_subcores=16)

    @functools.partial(
        pl.kernel,
        out_type=jax.ShapeDtypeStruct((NB * mid * NPAD,), _F32),
        mesh=mesh,
        compiler_params=pltpu.CompilerParams(needs_layout_passes=False),
        scratch_types=[
            pltpu.VMEM((cpt * NPAD,), _F32),
            pltpu.VMEM((cpt * NPAD,), _F32),
            pltpu.VMEM((3 * chunk,), jnp.int32),
            pltpu.VMEM((3 * chunk,), jnp.int32),
            pltpu.SemaphoreType.DMA,
            pltpu.SemaphoreType.DMA,
            pltpu.SemaphoreType.DMA,
        ],
    )
    def spmm(s_hbm, pk_hbm, zero_hbm, out_hbm,
             sup_v, agg_v, eb0, eb1, sem0, sem1, semz):
        b = lax.axis_index("c")
        t = lax.axis_index("s")
        base = (b * mid + t * cpt) * NPAD
        ebufs = (eb0, eb1)
        sems = (sem0, sem1)
        cps = pltpu.async_copy(s_hbm.at[pl.ds(base, cpt * NPAD)], sup_v, semz)
        cpz = pltpu.async_copy(zero_hbm.at[pl.ds(0, cpt * NPAD)], agg_v, semz)
        cp = [None] * _NCHUNK
        cp[0] = pltpu.async_copy(pk_hbm.at[pl.ds(0, 3 * chunk)], eb0, sem0)
        cps.wait()
        cpz.wait()

        for c in range(_NCHUNK):
            if c + 1 < _NCHUNK:
                cp[c + 1] = pltpu.async_copy(
                    pk_hbm.at[pl.ds((c + 1) * 3 * chunk, 3 * chunk)],
                    ebufs[(c + 1) % 2], sems[(c + 1) % 2])
            cp[c].wait()
            eb = ebufs[c % 2]

            @plsc.parallel_loop(0, chunk // 16, 1, unroll=8)
            def ebody(i, eb=eb):
                sl = pl.ds(i * 16, 16)
                si = eb[sl]
                di = eb[pl.ds(chunk + i * 16, 16)]
                wv = plsc.bitcast(eb[pl.ds(2 * chunk + i * 16, 16)], _F32)
                for ch in range(cpt):
                    off = ch * NPAD
                    vals = plsc.load_gather(sup_v, [si + off]) * wv
                    plsc.addupdate_scatter(agg_v, [di + off], vals)

        pltpu.sync_copy(agg_v, out_hbm.at[pl.ds(base, cpt * NPAD)])

    return spmm


def _prep_a(p):
    return dict(pre_g=p['pre_g'][:, None], pre_b=p['pre_b'][:, None],
                lin1_W=p['lin1_W'], lin1_b=p['lin1_b'][:, None],
                n1_g=p['n1_g'][:, None], n1_b=p['n1_b'][:, None],
                conv_Wt=p['conv_W'].T)


def _prep_b(p):
    d = dict(conv_b=p['conv_b'][:, None],
             n2_g=p['n2_g'][:, None], n2_b=p['n2_b'][:, None],
             lin2_W=p['lin2_W'], lin2_b=p['lin2_b'][:, None])
    if 'skip_W' in p:
        d['skip_W'] = p['skip_W']
        d['skip_b'] = p['skip_b'][:, None]
    return d


def kernel(image_enc, ref_vertices, edge_w, params, edge_src, edge_dst):
    mask = jnp.asarray(_MASK_NP)
    gnm = {}
    for c in (16, 32, 64, 128):
        gnm['gs%d' % c] = jnp.asarray(_GS_NP[c])
        gnm['et%d' % c] = jnp.asarray(_GS_NP[c].T)
    blocks = list(params['blocks']) + list(params['shape_blocks'])

    refp = jnp.zeros((8, NPAD), _F32).at[:3, :NV].set(ref_vertices)
    w_ref = jnp.zeros((128, 8), _F32).at[:, :3].set(params['lin0_W'][:, :3])
    w_img = params['lin0_W'][:, 3:]

    x, s = _run_tc(
        _k0_fn,
        dict(img=image_enc, refp=refp, Wref=w_ref, Wimg=w_img,
             b0=params['lin0_b'][:, None], mask=mask, gnm=gnm,
             pa=_prep_a(blocks[0])),
        [(NB, 128, NPAD), (NB, 32, NPAD)],
    )

    wbits = lax.bitcast_convert_type(edge_w, jnp.int32)
    pk = jnp.stack([edge_src.reshape(_NCHUNK, _CHUNK),
                    edge_dst.reshape(_NCHUNK, _CHUNK),
                    wbits.reshape(_NCHUNK, _CHUNK)], axis=1).reshape(-1)
    zero_buf = jnp.zeros((2 * NPAD,), _F32)

    for k in range(8):
        mid = blocks[k]['conv_W'].shape[0]
        g = _make_spmm(mid)(s.reshape(-1), pk, zero_buf)
        g = g.reshape(NB, mid, NPAD)
        if k < 7:
            mid_next = blocks[k + 1]['conv_W'].shape[0]
            out_c = blocks[k]['lin2_W'].shape[0]
            x, s = _run_tc(
                _kmid_fn,
                dict(x=x, g=g, mask=mask, gnm=gnm,
                     pb=_prep_b(blocks[k]), pa=_prep_a(blocks[k + 1])),
                [(NB, out_c, NPAD), (NB, mid_next, NPAD)],
            )
        else:
            out, = _run_tc(
                _kfin_fn,
                dict(x=x, g=g, mask=mask, gnm=gnm, pb=_prep_b(blocks[k]),
                     gn_g=params['gn_g'][:, None], gn_b=params['gn_b'][:, None],
                     Wout=params['lin_out_W'], bout=params['lin_out_b'][:, None]),
                [(NB, 3, NPAD)],
            )

    return jnp.transpose(out, (0, 2, 1))[:, :NV, :]


# speedup vs baseline: 1.1934x; 1.1895x over previous
import functools

import numpy as np
import jax
import jax.numpy as jnp
from jax import lax
from jax.experimental import pallas as pl
from jax.experimental.pallas import tpu as pltpu
from jax.experimental.pallas import tpu_sc as plsc

NV = 6890
NPAD = 6912
NEDGE = 6890 * 8
NB = 2
EPS = 1e-5
CNT = 8.0 * NV

_F32 = jnp.float32

_GS_NP = {}
for _c in (16, 32, 64, 128):
    _g = np.zeros((_c // 8, _c), np.float32)
    for _i in range(_c // 8):
        _g[_i, 8 * _i:8 * _i + 8] = 1.0
    _GS_NP[_c] = _g

_MASK_NP = np.zeros((1, NPAD), np.float32)
_MASK_NP[0, :NV] = 1.0


def _mm(a, b):
    return lax.dot_general(a, b, (((1,), (0,)), ((), ())),
                           preferred_element_type=_F32)


def _gn_relu(x, gvec, bvec, mask, gnm):
    del gnm
    c = x.shape[0]

    def gsum_exp(s):
        parts = []
        for g in range(c // 8):
            seg = jnp.sum(s[8 * g:8 * g + 8], axis=0, keepdims=True)
            parts.append(jnp.broadcast_to(seg, (8, 1)))
        return jnp.concatenate(parts, axis=0)

    s1 = jnp.sum(x, axis=1, keepdims=True)
    mc = gsum_exp(s1) / CNT
    d = (x - mc) * mask
    s2 = jnp.sum(d * d, axis=1, keepdims=True)
    vc = gsum_exp(s2) / CNT
    xn = d / jnp.sqrt(vc + EPS)
    return jnp.maximum(xn * gvec + bvec, 0.0) * mask


def _part_a(x, pa, mask, gnm):
    y = _gn_relu(x, pa['pre_g'], pa['pre_b'], mask, gnm)
    y = (_mm(pa['lin1_W'], y) + pa['lin1_b']) * mask
    y = _gn_relu(y, pa['n1_g'], pa['n1_b'], mask, gnm)
    return _mm(pa['conv_Wt'], y)


def _part_b(x, g, pb, mask, gnm):
    t = (g + pb['conv_b']) * mask
    z = _gn_relu(t, pb['n2_g'], pb['n2_b'], mask, gnm)
    y2 = (_mm(pb['lin2_W'], z) + pb['lin2_b']) * mask
    if 'skip_W' in pb:
        x = (_mm(pb['skip_W'], x) + pb['skip_b']) * mask
    return x + y2


def _run_tc(fn, inputs, out_shapes):
    flat, tdef = jax.tree_util.tree_flatten(inputs)
    n_in = len(flat)

    def body(*refs):
        ins = jax.tree_util.tree_unflatten(tdef, refs[:n_in])
        fn(ins, refs[n_in:])

    return pl.pallas_call(
        body,
        out_shape=[jax.ShapeDtypeStruct(s, _F32) for s in out_shapes],
    )(*flat)


def _k0_fn(ins, outs):
    mask = ins['mask'][...]
    gnm = {k: ins['gnm'][k][...] for k in ins['gnm']}
    refpart = _mm(ins['Wref'][...], ins['refp'][...])
    imgproj = lax.dot_general(ins['Wimg'][...], ins['img'][...],
                              (((1,), (1,)), ((), ())),
                              preferred_element_type=_F32)
    pa = {k: ins['pa'][k][...] for k in ins['pa']}
    for b in range(NB):
        x0 = (refpart + imgproj[:, b:b + 1] + ins['b0'][...]) * mask
        outs[0][b] = x0
        outs[1][b] = _part_a(x0, pa, mask, gnm)


def _kmid_fn(ins, outs):
    mask = ins['mask'][...]
    gnm = {k: ins['gnm'][k][...] for k in ins['gnm']}
    pb = {k: ins['pb'][k][...] for k in ins['pb']}
    pa = {k: ins['pa'][k][...] for k in ins['pa']}
    for b in range(NB):
        xk = _part_b(ins['x'][b], ins['g'][b], pb, mask, gnm)
        outs[0][b] = xk
        outs[1][b] = _part_a(xk, pa, mask, gnm)


def _kfin_fn(ins, outs):
    mask = ins['mask'][...]
    gnm = {k: ins['gnm'][k][...] for k in ins['gnm']}
    pb = {k: ins['pb'][k][...] for k in ins['pb']}
    for b in range(NB):
        xk = _part_b(ins['x'][b], ins['g'][b], pb, mask, gnm)
        z = _gn_relu(xk, ins['gn_g'][...], ins['gn_b'][...], mask, gnm)
        outs[0][b] = _mm(ins['Wout'][...], z) + ins['bout'][...]


_CHUNK = 11024
_NCHUNK = NEDGE // _CHUNK


@functools.cache
def _make_spmm(mid):
    cpt = mid // 16
    chunk = _CHUNK
    mesh = plsc.VectorSubcoreMesh(core_axis_name="c", subcore_axis_name="s",
                                  num_cores=2, num_subcores=16)

    @functools.partial(
        pl.kernel,
        out_type=jax.ShapeDtypeStruct((NB * mid * NPAD,), _F32),
        mesh=mesh,
        compiler_params=pltpu.CompilerParams(needs_layout_passes=False),
        scratch_types=[
            pltpu.VMEM((cpt * NPAD,), _F32),
            pltpu.VMEM((cpt * NPAD,), _F32),
            pltpu.VMEM((3 * chunk,), jnp.int32),
            pltpu.VMEM((3 * chunk,), jnp.int32),
            pltpu.SemaphoreType.DMA,
            pltpu.SemaphoreType.DMA,
            pltpu.SemaphoreType.DMA,
        ],
    )
    def spmm(s_hbm, pk_hbm, zero_hbm, out_hbm,
             sup_v, agg_v, eb0, eb1, sem0, sem1, semz):
        b = lax.axis_index("c")
        t = lax.axis_index("s")
        base = (b * mid + t * cpt) * NPAD
        ebufs = (eb0, eb1)
        sems = (sem0, sem1)
        cps = pltpu.async_copy(s_hbm.at[pl.ds(base, cpt * NPAD)], sup_v, semz)
        cpz = pltpu.async_copy(zero_hbm.at[pl.ds(0, cpt * NPAD)], agg_v, semz)
        cp = [None] * _NCHUNK
        cp[0] = pltpu.async_copy(pk_hbm.at[pl.ds(0, 3 * chunk)], eb0, sem0)
        cps.wait()
        cpz.wait()

        for c in range(_NCHUNK):
            if c + 1 < _NCHUNK:
                cp[c + 1] = pltpu.async_copy(
                    pk_hbm.at[pl.ds((c + 1) * 3 * chunk, 3 * chunk)],
                    ebufs[(c + 1) % 2], sems[(c + 1) % 2])
            cp[c].wait()
            eb = ebufs[c % 2]

            @plsc.parallel_loop(0, chunk // 16, 1, unroll=8)
            def ebody(i, eb=eb):
                sl = pl.ds(i * 16, 16)
                si = eb[sl]
                di = eb[pl.ds(chunk + i * 16, 16)]
                wv = plsc.bitcast(eb[pl.ds(2 * chunk + i * 16, 16)], _F32)
                for ch in range(cpt):
                    off = ch * NPAD
                    vals = plsc.load_gather(sup_v, [si + off]) * wv
                    plsc.addupdate_scatter(agg_v, [di + off], vals)

        pltpu.sync_copy(agg_v, out_hbm.at[pl.ds(base, cpt * NPAD)])

    return spmm


def _prep_a(p):
    return dict(pre_g=p['pre_g'][:, None], pre_b=p['pre_b'][:, None],
                lin1_W=p['lin1_W'], lin1_b=p['lin1_b'][:, None],
                n1_g=p['n1_g'][:, None], n1_b=p['n1_b'][:, None],
                conv_Wt=p['conv_W'].T)


def _prep_b(p):
    d = dict(conv_b=p['conv_b'][:, None],
             n2_g=p['n2_g'][:, None], n2_b=p['n2_b'][:, None],
             lin2_W=p['lin2_W'], lin2_b=p['lin2_b'][:, None])
    if 'skip_W' in p:
        d['skip_W'] = p['skip_W']
        d['skip_b'] = p['skip_b'][:, None]
    return d


def kernel(image_enc, ref_vertices, edge_w, params, edge_src, edge_dst):
    mask = jnp.asarray(_MASK_NP)
    gnm = {}
    for c in (16, 32, 64, 128):
        gnm['gs%d' % c] = jnp.asarray(_GS_NP[c])
        gnm['et%d' % c] = jnp.asarray(_GS_NP[c].T)
    blocks = list(params['blocks']) + list(params['shape_blocks'])

    refp = jnp.zeros((8, NPAD), _F32).at[:3, :NV].set(ref_vertices)
    w_ref = jnp.zeros((128, 8), _F32).at[:, :3].set(params['lin0_W'][:, :3])
    w_img = params['lin0_W'][:, 3:]

    x, s = _run_tc(
        _k0_fn,
        dict(img=image_enc, refp=refp, Wref=w_ref, Wimg=w_img,
             b0=params['lin0_b'][:, None], mask=mask, gnm=gnm,
             pa=_prep_a(blocks[0])),
        [(NB, 128, NPAD), (NB, 32, NPAD)],
    )

    wbits = lax.bitcast_convert_type(edge_w, jnp.int32)
    pk = jnp.stack([edge_src.reshape(_NCHUNK, _CHUNK),
                    edge_dst.reshape(_NCHUNK, _CHUNK),
                    wbits.reshape(_NCHUNK, _CHUNK)], axis=1).reshape(-1)
    zero_buf = jnp.zeros((2 * NPAD,), _F32)

    for k in range(8):
        mid = blocks[k]['conv_W'].shape[0]
        g = _make_spmm(mid)(s.reshape(-1), pk, zero_buf)
        g = g.reshape(NB, mid, NPAD)
        if k < 7:
            mid_next = blocks[k + 1]['conv_W'].shape[0]
            out_c = blocks[k]['lin2_W'].shape[0]
            x, s = _run_tc(
                _kmid_fn,
                dict(x=x, g=g, mask=mask, gnm=gnm,
                     pb=_prep_b(blocks[k]), pa=_prep_a(blocks[k + 1])),
                [(NB, out_c, NPAD), (NB, mid_next, NPAD)],
            )
        else:
            out, = _run_tc(
                _kfin_fn,
                dict(x=x, g=g, mask=mask, gnm=gnm, pb=_prep_b(blocks[k]),
                     gn_g=params['gn_g'][:, None], gn_b=params['gn_b'][:, None],
                     Wout=params['lin_out_W'], bout=params['lin_out_b'][:, None]),
                [(NB, 3, NPAD)],
            )

    return jnp.transpose(out, (0, 2, 1))[:, :NV, :]
